# baseline (speedup 1.0000x reference)
import functools

import jax
import jax.numpy as jnp
from jax import lax
from jax.experimental import pallas as pl
from jax.experimental.pallas import tpu as pltpu
from jax.experimental.pallas import tpu_sc as plsc

N_NODES = 10000
N_EDGES = 160000
IN_DIM = 256
HID_DIM = 512
OUT_DIM = 256

NC = 2
NS = 16
L = 16

COLS = 128
CHUNK = 128
E_PAD = 163840
E_PER_TILE = E_PAD // NS
N_CHUNKS = E_PER_TILE // CHUNK
E_PER_HTILE = E_PAD // (NC * NS)
NPAD = 10240
TRASH = N_NODES
NACC = 10016
ACC_PER_TILE = NACC // NS
MB = 1024
N_MBLOCKS = -(-N_NODES // MB)


def _zero_1d(ref, n):
    z = jnp.zeros((L,), jnp.float32)

    def body(i, c):
        ref[pl.ds(i * L, L)] = z
        return c

    lax.fori_loop(0, n // L, body, 0)


def _zero_2d(ref, nrows, ncols):
    z = jnp.zeros((L,), jnp.float32)
    per_row = ncols // L

    def body(i, c):
        ref[i // per_row, pl.ds((i % per_row) * L, L)] = z
        return c

    lax.fori_loop(0, nrows * per_row, body, 0)


@functools.cache
def _sc_mesh():
    return plsc.VectorSubcoreMesh(core_axis_name="c", subcore_axis_name="s")


@functools.cache
def _deg_hist_kernel():
    return functools.partial(
        pl.kernel,
        out_type=jax.ShapeDtypeStruct((NC * NS, NPAD), jnp.float32),
        mesh=_sc_mesh(),
        compiler_params=pltpu.CompilerParams(needs_layout_passes=False, use_tc_tiling_on_sc=False),
        scratch_types=[
            pltpu.VMEM((E_PER_HTILE,), jnp.int32),
            pltpu.VMEM((NPAD,), jnp.float32),
        ],
    )(_deg_hist)


def _deg_hist(dst_hbm, hist_hbm, idx_v, hist_v):
    cid = lax.axis_index("c")
    sid = lax.axis_index("s")
    wid = cid * NS + sid
    pltpu.sync_copy(dst_hbm.at[pl.ds(wid * E_PER_HTILE, E_PER_HTILE)], idx_v)
    _zero_1d(hist_v, NPAD)
    ones = jnp.ones((L,), jnp.float32)

    def body(i, c):
        idx = idx_v[pl.ds(i * L, L)]
        plsc.addupdate_scatter(hist_v, [idx], ones)
        return c

    lax.fori_loop(0, E_PER_HTILE // L, body, 0)
    pltpu.sync_copy(hist_v, hist_hbm.at[wid])


NROW = 3
NIDX = 5
UNROLL = 15


@functools.cache
def _edge_agg_kernel():
    return functools.partial(
        pl.kernel,
        out_type=(
            jax.ShapeDtypeStruct((N_NODES, COLS), jnp.float32),
            jax.ShapeDtypeStruct((N_NODES, COLS), jnp.float32),
        ),
        mesh=_sc_mesh(),
        compiler_params=pltpu.CompilerParams(needs_layout_passes=False, use_tc_tiling_on_sc=False),
        scratch_types=[pltpu.VMEM((CHUNK, COLS), jnp.float32) for _ in range(NROW)]
        + [pltpu.VMEM((CHUNK,), jnp.int32) for _ in range(2 * NIDX)]
        + [pltpu.SemaphoreType.DMA for _ in range(2 * NROW + NIDX)]
        + [pltpu.VMEM_SHARED((NACC, COLS), jnp.float32)],
    )(_edge_agg)


def _edge_agg(vs_l_hbm, vs_r_hbm, src_hbm, dst_hbm, acc_l_hbm, acc_r_hbm,
              *scratch):
    rows = scratch[:NROW]
    isrc = scratch[NROW:NROW + NIDX]
    idst = scratch[NROW + NIDX:NROW + 2 * NIDX]
    gsem = scratch[NROW + 2 * NIDX:2 * NROW + 2 * NIDX]
    ssem = scratch[2 * NROW + 2 * NIDX:3 * NROW + 2 * NIDX]
    isem = scratch[3 * NROW + 2 * NIDX:3 * NROW + 3 * NIDX]
    acc_sh = scratch[3 * NROW + 3 * NIDX]
    cid = lax.axis_index("c")
    sid = lax.axis_index("s")
    chunk0 = sid * N_CHUNKS

    _zero_2d(rows[0], CHUNK, COLS)
    arow0 = sid * ACC_PER_TILE
    for k in range(ACC_PER_TILE // CHUNK):
        pltpu.sync_copy(rows[0], acc_sh.at[pl.ds(arow0 + k * CHUNK, CHUNK)])
    rem = ACC_PER_TILE % CHUNK
    if rem:
        pltpu.sync_copy(
            rows[0].at[pl.ds(0, rem)],
            acc_sh.at[pl.ds(arow0 + (ACC_PER_TILE // CHUNK) * CHUNK, rem)],
        )
    plsc.subcore_barrier()

    def _idx_fetch(c, k):
        pltpu.async_copy(src_hbm.at[chunk0 + c], isrc[k], isem[k])
        pltpu.async_copy(dst_hbm.at[chunk0 + c], idst[k], isem[k])

    def _idx_wait(c, k):
        pltpu.make_async_copy(src_hbm.at[chunk0 + c], isrc[k], isem[k]).wait()
        pltpu.make_async_copy(dst_hbm.at[chunk0 + c], idst[k], isem[k]).wait()

    def _gather_start(b, k):
        @pl.when(cid == 0)
        def _():
            pltpu.async_copy(vs_l_hbm.at[isrc[k]], rows[b], gsem[b])

        @pl.when(cid == 1)
        def _():
            pltpu.async_copy(vs_r_hbm.at[isrc[k]], rows[b], gsem[b])

    def _gather_wait(b, k):
        @pl.when(cid == 0)
        def _():
            pltpu.make_async_copy(vs_l_hbm.at[isrc[k]], rows[b], gsem[b]).wait()

        @pl.when(cid == 1)
        def _():
            pltpu.make_async_copy(vs_r_hbm.at[isrc[k]], rows[b], gsem[b]).wait()

    def _scatter_start(b, k):
        pltpu.async_copy(rows[b], acc_sh.at[idst[k]], ssem[b], add=True)

    def _scatter_wait(b, k):
        pltpu.make_async_copy(rows[b], acc_sh.at[idst[k]], ssem[b]).wait()

    def _cycle(c, j, do_swait, do_gather, do_fetch):
        if do_swait:
            _scatter_wait((j - 1) % NROW, (j - 1) % NIDX)
        if do_gather:
            _idx_wait(c + 2, (j + 2) % NIDX)
            _gather_start((j + 2) % NROW, (j + 2) % NIDX)
        _gather_wait(j % NROW, j % NIDX)
        _scatter_start(j % NROW, j % NIDX)
        if do_fetch:
            _idx_fetch(c + 4, (j + 4) % NIDX)

    for k in range(4):
        _idx_fetch(k, k)
    for c in range(2):
        _idx_wait(c, c)
        _gather_start(c, c)
    _cycle(0, 0, do_swait=False, do_gather=True, do_fetch=True)

    def round_body(r, carry):
        c0 = 1 + r * UNROLL
        for j in range(UNROLL):
            _cycle(c0 + j, 1 + j, do_swait=True, do_gather=True, do_fetch=True)
        return carry

    lax.fori_loop(0, (N_CHUNKS - 5) // UNROLL, round_body, 0)
    for c in range(N_CHUNKS - 4, N_CHUNKS):
        _cycle(c, c, do_swait=True, do_gather=(c + 2 < N_CHUNKS),
               do_fetch=False)
    _scatter_wait((N_CHUNKS - 1) % NROW, (N_CHUNKS - 1) % NIDX)
    plsc.subcore_barrier()

    orow0 = sid * (N_NODES // NS)

    @pl.when(cid == 0)
    def _():
        pltpu.sync_copy(acc_sh.at[pl.ds(orow0, N_NODES // NS)],
                        acc_l_hbm.at[pl.ds(orow0, N_NODES // NS)])

    @pl.when(cid == 1)
    def _():
        pltpu.sync_copy(acc_sh.at[pl.ds(orow0, N_NODES // NS)],
                        acc_r_hbm.at[pl.ds(orow0, N_NODES // NS)])


def _pre_body(x_ref, h_ref, vl_ref, vr_ref, dis_ref):
    deg = jnp.sum(h_ref[...], axis=0) + 1.0
    dis = lax.rsqrt(deg)
    vs = x_ref[...] * dis[:, None]
    vl_ref[...] = vs[:, :COLS]
    vr_ref[...] = vs[:, COLS:]
    dis_ref[...] = dis[:, None]


_pre = pl.pallas_call(
    _pre_body,
    grid=(N_MBLOCKS,),
    in_specs=[
        pl.BlockSpec((MB, IN_DIM), lambda i: (i, 0)),
        pl.BlockSpec((NC * NS, MB), lambda i: (0, i)),
    ],
    out_specs=[
        pl.BlockSpec((MB, COLS), lambda i: (i, 0)),
        pl.BlockSpec((MB, COLS), lambda i: (i, 0)),
        pl.BlockSpec((MB, 1), lambda i: (i, 0)),
    ],
    out_shape=[
        jax.ShapeDtypeStruct((N_NODES, COLS), jnp.float32),
        jax.ShapeDtypeStruct((N_NODES, COLS), jnp.float32),
        jax.ShapeDtypeStruct((N_NODES, 1), jnp.float32),
    ],
)


def _mid_body(vl_ref, vr_ref, al_ref, ar_ref, dis_ref, W1_ref, b1_ref, W2_ref,
              ol_ref, or_ref):
    d = dis_ref[...]
    y = jnp.concatenate(
        [al_ref[...] + vl_ref[...], ar_ref[...] + vr_ref[...]], axis=1) * d
    h = jnp.maximum(
        jnp.dot(y.astype(jnp.bfloat16), W1_ref[...],
                preferred_element_type=jnp.float32) + b1_ref[...], 0.0)
    z = jnp.dot(h.astype(jnp.bfloat16), W2_ref[...],
                preferred_element_type=jnp.float32) * d
    ol_ref[...] = z[:, :COLS]
    or_ref[...] = z[:, COLS:]


_mid = pl.pallas_call(
    _mid_body,
    grid=(N_MBLOCKS,),
    in_specs=[
        pl.BlockSpec((MB, COLS), lambda i: (i, 0)),
        pl.BlockSpec((MB, COLS), lambda i: (i, 0)),
        pl.BlockSpec((MB, COLS), lambda i: (i, 0)),
        pl.BlockSpec((MB, COLS), lambda i: (i, 0)),
        pl.BlockSpec((MB, 1), lambda i: (i, 0)),
        pl.BlockSpec((IN_DIM, HID_DIM), lambda i: (0, 0)),
        pl.BlockSpec((1, HID_DIM), lambda i: (0, 0)),
        pl.BlockSpec((HID_DIM, OUT_DIM), lambda i: (0, 0)),
    ],
    out_specs=[
        pl.BlockSpec((MB, COLS), lambda i: (i, 0)),
        pl.BlockSpec((MB, COLS), lambda i: (i, 0)),
    ],
    out_shape=[
        jax.ShapeDtypeStruct((N_NODES, COLS), jnp.float32),
        jax.ShapeDtypeStruct((N_NODES, COLS), jnp.float32),
    ],
)


def _post_body(vl_ref, vr_ref, al_ref, ar_ref, dis_ref, b2_ref, o_ref):
    d = dis_ref[...]
    y = jnp.concatenate(
        [al_ref[...] + vl_ref[...], ar_ref[...] + vr_ref[...]], axis=1) * d
    o_ref[...] = jax.nn.sigmoid(y + b2_ref[...])


_post = pl.pallas_call(
    _post_body,
    grid=(N_MBLOCKS,),
    in_specs=[
        pl.BlockSpec((MB, COLS), lambda i: (i, 0)),
        pl.BlockSpec((MB, COLS), lambda i: (i, 0)),
        pl.BlockSpec((MB, COLS), lambda i: (i, 0)),
        pl.BlockSpec((MB, COLS), lambda i: (i, 0)),
        pl.BlockSpec((MB, 1), lambda i: (i, 0)),
        pl.BlockSpec((1, OUT_DIM), lambda i: (0, 0)),
    ],
    out_specs=pl.BlockSpec((MB, OUT_DIM), lambda i: (i, 0)),
    out_shape=jax.ShapeDtypeStruct((N_NODES, OUT_DIM), jnp.float32),
)


def kernel(x, edge_index, batch, W1, b1, W2, b2):
    ei = edge_index.astype(jnp.int32)
    npad = E_PAD - N_EDGES
    src = jnp.concatenate([ei[0], jnp.zeros((npad,), jnp.int32)])
    dst = jnp.concatenate([ei[1], jnp.full((npad,), TRASH, jnp.int32)])
    src2d = src.reshape(E_PAD // CHUNK, CHUNK)
    dst2d = dst.reshape(E_PAD // CHUNK, CHUNK)

    hists = _deg_hist_kernel()(dst)
    vs_l, vs_r, dis = _pre(x, hists)
    acc1_l, acc1_r = _edge_agg_kernel()(vs_l, vs_r, src2d, dst2d)
    vs2_l, vs2_r = _mid(vs_l, vs_r, acc1_l, acc1_r, dis,
                        W1.astype(jnp.bfloat16), b1.reshape(1, -1),
                        W2.astype(jnp.bfloat16))
    acc2_l, acc2_r = _edge_agg_kernel()(vs2_l, vs2_r, src2d, dst2d)
    return _post(vs2_l, vs2_r, acc2_l, acc2_r, dis, b2.reshape(1, -1))


# speedup vs baseline: 1.0950x; 1.0950x over previous
import functools

import jax
import jax.numpy as jnp
from jax import lax
from jax.experimental import pallas as pl
from jax.experimental.pallas import tpu as pltpu
from jax.experimental.pallas import tpu_sc as plsc

N_NODES = 10000
N_EDGES = 160000
IN_DIM = 256
HID_DIM = 512
OUT_DIM = 256

NC = 2
NS = 16
L = 16

COLS = 128
CHUNK = 128
E_PAD = 163840
E_PER_TILE = E_PAD // NS
N_CHUNKS = E_PER_TILE // CHUNK
E_PER_HTILE = E_PAD // (NC * NS)
NPAD = 10240
TRASH = N_NODES
NACC = 10016
ACC_PER_TILE = NACC // NS
MB = 1024
N_MBLOCKS = -(-N_NODES // MB)


def _zero_1d(ref, n):
    z = jnp.zeros((L,), jnp.float32)

    def body(i, c):
        ref[pl.ds(i * L, L)] = z
        return c

    lax.fori_loop(0, n // L, body, 0)


def _zero_2d(ref, nrows, ncols):
    z = jnp.zeros((L,), jnp.float32)
    per_row = ncols // L

    def body(i, c):
        ref[i // per_row, pl.ds((i % per_row) * L, L)] = z
        return c

    lax.fori_loop(0, nrows * per_row, body, 0)


@functools.cache
def _sc_mesh():
    return plsc.VectorSubcoreMesh(core_axis_name="c", subcore_axis_name="s")


@functools.cache
def _deg_hist_kernel():
    return functools.partial(
        pl.kernel,
        out_type=jax.ShapeDtypeStruct((NC * NS, NPAD), jnp.float32),
        mesh=_sc_mesh(),
        compiler_params=pltpu.CompilerParams(needs_layout_passes=False, use_tc_tiling_on_sc=False),
        scratch_types=[
            pltpu.VMEM((E_PER_HTILE,), jnp.int32),
            pltpu.VMEM((NPAD,), jnp.float32),
        ],
    )(_deg_hist)


def _deg_hist(dst_hbm, hist_hbm, idx_v, hist_v):
    cid = lax.axis_index("c")
    sid = lax.axis_index("s")
    wid = cid * NS + sid
    pltpu.sync_copy(dst_hbm.at[pl.ds(wid * E_PER_HTILE, E_PER_HTILE)], idx_v)
    _zero_1d(hist_v, NPAD)
    ones = jnp.ones((L,), jnp.float32)

    def body(i, c):
        idx = idx_v[pl.ds(i * L, L)]
        plsc.addupdate_scatter(hist_v, [idx], ones)
        return c

    lax.fori_loop(0, E_PER_HTILE // L, body, 0)
    pltpu.sync_copy(hist_v, hist_hbm.at[wid])


NROW = 3
NIDX = 5
UNROLL = 15


@functools.cache
def _edge_agg_kernel():
    return functools.partial(
        pl.kernel,
        out_type=(
            jax.ShapeDtypeStruct((N_NODES, COLS), jnp.float32),
            jax.ShapeDtypeStruct((N_NODES, COLS), jnp.float32),
        ),
        mesh=_sc_mesh(),
        compiler_params=pltpu.CompilerParams(needs_layout_passes=False, use_tc_tiling_on_sc=False),
        scratch_types=[pltpu.VMEM((CHUNK, COLS), jnp.float32) for _ in range(NROW)]
        + [pltpu.VMEM((CHUNK,), jnp.int32) for _ in range(2 * NIDX)]
        + [pltpu.SemaphoreType.DMA for _ in range(2 * NROW + NIDX)]
        + [pltpu.VMEM_SHARED((NACC, COLS), jnp.float32)],
    )(_edge_agg)


def _edge_agg(vs_l_hbm, vs_r_hbm, src_hbm, dst_hbm, acc_l_hbm, acc_r_hbm,
              *scratch):
    rows = scratch[:NROW]
    isrc = scratch[NROW:NROW + NIDX]
    idst = scratch[NROW + NIDX:NROW + 2 * NIDX]
    gsem = scratch[NROW + 2 * NIDX:2 * NROW + 2 * NIDX]
    ssem = scratch[2 * NROW + 2 * NIDX:3 * NROW + 2 * NIDX]
    isem = scratch[3 * NROW + 2 * NIDX:3 * NROW + 3 * NIDX]
    acc_sh = scratch[3 * NROW + 3 * NIDX]
    cid = lax.axis_index("c")
    sid = lax.axis_index("s")
    chunk0 = sid * N_CHUNKS

    _zero_2d(rows[0], CHUNK, COLS)
    arow0 = sid * ACC_PER_TILE
    for k in range(ACC_PER_TILE // CHUNK):
        pltpu.sync_copy(rows[0], acc_sh.at[pl.ds(arow0 + k * CHUNK, CHUNK)])
    rem = ACC_PER_TILE % CHUNK
    if rem:
        pltpu.sync_copy(
            rows[0].at[pl.ds(0, rem)],
            acc_sh.at[pl.ds(arow0 + (ACC_PER_TILE // CHUNK) * CHUNK, rem)],
        )
    plsc.subcore_barrier()

    def _idx_fetch(c, k):
        pltpu.async_copy(src_hbm.at[chunk0 + c], isrc[k], isem[k])
        pltpu.async_copy(dst_hbm.at[chunk0 + c], idst[k], isem[k])

    def _idx_wait(c, k):
        pltpu.make_async_copy(src_hbm.at[chunk0 + c], isrc[k], isem[k]).wait()
        pltpu.make_async_copy(dst_hbm.at[chunk0 + c], idst[k], isem[k]).wait()

    def _gather_start(b, k):
        @pl.when(cid == 0)
        def _():
            pltpu.async_copy(vs_l_hbm.at[isrc[k]], rows[b], gsem[b])

        @pl.when(cid == 1)
        def _():
            pltpu.async_copy(vs_r_hbm.at[isrc[k]], rows[b], gsem[b])

    def _gather_wait(b, k):
        @pl.when(cid == 0)
        def _():
            pltpu.make_async_copy(vs_l_hbm.at[isrc[k]], rows[b], gsem[b]).wait()

        @pl.when(cid == 1)
        def _():
            pltpu.make_async_copy(vs_r_hbm.at[isrc[k]], rows[b], gsem[b]).wait()

    def _scatter_start(b, k):
        pltpu.async_copy(rows[b], acc_sh.at[idst[k]], ssem[b], add=True)

    def _scatter_wait(b, k):
        pltpu.make_async_copy(rows[b], acc_sh.at[idst[k]], ssem[b]).wait()

    def _cycle(c, j, do_swait, do_gather, do_fetch):
        if do_swait:
            _scatter_wait((j - 1) % NROW, (j - 1) % NIDX)
        if do_gather:
            _idx_wait(c + 2, (j + 2) % NIDX)
            _gather_start((j + 2) % NROW, (j + 2) % NIDX)
        _gather_wait(j % NROW, j % NIDX)
        _scatter_start(j % NROW, j % NIDX)
        if do_fetch:
            _idx_fetch(c + 4, (j + 4) % NIDX)

    for k in range(4):
        _idx_fetch(k, k)
    for c in range(2):
        _idx_wait(c, c)
        _gather_start(c, c)
    _cycle(0, 0, do_swait=False, do_gather=True, do_fetch=True)

    def round_body(r, carry):
        c0 = 1 + r * UNROLL
        for j in range(UNROLL):
            _cycle(c0 + j, 1 + j, do_swait=True, do_gather=True, do_fetch=True)
        return carry

    lax.fori_loop(0, (N_CHUNKS - 5) // UNROLL, round_body, 0)
    for c in range(N_CHUNKS - 4, N_CHUNKS):
        _cycle(c, c, do_swait=True, do_gather=(c + 2 < N_CHUNKS),
               do_fetch=False)
    _scatter_wait((N_CHUNKS - 1) % NROW, (N_CHUNKS - 1) % NIDX)
    plsc.subcore_barrier()

    orow0 = sid * (N_NODES // NS)

    @pl.when(cid == 0)
    def _():
        pltpu.sync_copy(acc_sh.at[pl.ds(orow0, N_NODES // NS)],
                        acc_l_hbm.at[pl.ds(orow0, N_NODES // NS)])

    @pl.when(cid == 1)
    def _():
        pltpu.sync_copy(acc_sh.at[pl.ds(orow0, N_NODES // NS)],
                        acc_r_hbm.at[pl.ds(orow0, N_NODES // NS)])


def _pre_body(x_ref, h_ref, vl_ref, vr_ref, dis_ref):
    deg = jnp.sum(h_ref[...], axis=0) + 1.0
    dis = lax.rsqrt(deg)
    vs = x_ref[...] * dis[:, None]
    vl_ref[...] = vs[:, :COLS]
    vr_ref[...] = vs[:, COLS:]
    dis_ref[...] = dis[:, None]


_pre = pl.pallas_call(
    _pre_body,
    grid=(N_MBLOCKS,),
    in_specs=[
        pl.BlockSpec((MB, IN_DIM), lambda i: (i, 0)),
        pl.BlockSpec((NC * NS, MB), lambda i: (0, i)),
    ],
    out_specs=[
        pl.BlockSpec((MB, COLS), lambda i: (i, 0)),
        pl.BlockSpec((MB, COLS), lambda i: (i, 0)),
        pl.BlockSpec((MB, 1), lambda i: (i, 0)),
    ],
    out_shape=[
        jax.ShapeDtypeStruct((N_NODES, COLS), jnp.float32),
        jax.ShapeDtypeStruct((N_NODES, COLS), jnp.float32),
        jax.ShapeDtypeStruct((N_NODES, 1), jnp.float32),
    ],
)


def _mid_body(vl_ref, vr_ref, al_ref, ar_ref, dis_ref, W1_ref, b1_ref, W2_ref,
              ol_ref, or_ref):
    d = dis_ref[...]
    y = jnp.concatenate(
        [al_ref[...] + vl_ref[...], ar_ref[...] + vr_ref[...]], axis=1) * d
    h = jnp.maximum(
        jnp.dot(y, W1_ref[...], preferred_element_type=jnp.float32)
        + b1_ref[...], 0.0)
    z = jnp.dot(h, W2_ref[...], preferred_element_type=jnp.float32) * d
    ol_ref[...] = z[:, :COLS]
    or_ref[...] = z[:, COLS:]


_mid = pl.pallas_call(
    _mid_body,
    grid=(N_MBLOCKS,),
    in_specs=[
        pl.BlockSpec((MB, COLS), lambda i: (i, 0)),
        pl.BlockSpec((MB, COLS), lambda i: (i, 0)),
        pl.BlockSpec((MB, COLS), lambda i: (i, 0)),
        pl.BlockSpec((MB, COLS), lambda i: (i, 0)),
        pl.BlockSpec((MB, 1), lambda i: (i, 0)),
        pl.BlockSpec((IN_DIM, HID_DIM), lambda i: (0, 0)),
        pl.BlockSpec((1, HID_DIM), lambda i: (0, 0)),
        pl.BlockSpec((HID_DIM, OUT_DIM), lambda i: (0, 0)),
    ],
    out_specs=[
        pl.BlockSpec((MB, COLS), lambda i: (i, 0)),
        pl.BlockSpec((MB, COLS), lambda i: (i, 0)),
    ],
    out_shape=[
        jax.ShapeDtypeStruct((N_NODES, COLS), jnp.float32),
        jax.ShapeDtypeStruct((N_NODES, COLS), jnp.float32),
    ],
)


def _post_body(vl_ref, vr_ref, al_ref, ar_ref, dis_ref, b2_ref, o_ref):
    d = dis_ref[...]
    y = jnp.concatenate(
        [al_ref[...] + vl_ref[...], ar_ref[...] + vr_ref[...]], axis=1) * d
    o_ref[...] = jax.nn.sigmoid(y + b2_ref[...])


_post = pl.pallas_call(
    _post_body,
    grid=(N_MBLOCKS,),
    in_specs=[
        pl.BlockSpec((MB, COLS), lambda i: (i, 0)),
        pl.BlockSpec((MB, COLS), lambda i: (i, 0)),
        pl.BlockSpec((MB, COLS), lambda i: (i, 0)),
        pl.BlockSpec((MB, COLS), lambda i: (i, 0)),
        pl.BlockSpec((MB, 1), lambda i: (i, 0)),
        pl.BlockSpec((1, OUT_DIM), lambda i: (0, 0)),
    ],
    out_specs=pl.BlockSpec((MB, OUT_DIM), lambda i: (i, 0)),
    out_shape=jax.ShapeDtypeStruct((N_NODES, OUT_DIM), jnp.float32),
)


def kernel(x, edge_index, batch, W1, b1, W2, b2):
    ei = edge_index.astype(jnp.int32)
    npad = E_PAD - N_EDGES
    src = jnp.concatenate([ei[0], jnp.zeros((npad,), jnp.int32)])
    dst = jnp.concatenate([ei[1], jnp.full((npad,), TRASH, jnp.int32)])
    src2d = src.reshape(E_PAD // CHUNK, CHUNK)
    dst2d = dst.reshape(E_PAD // CHUNK, CHUNK)

    hists = _deg_hist_kernel()(dst)
    vs_l, vs_r, dis = _pre(x, hists)
    acc1_l, acc1_r = _edge_agg_kernel()(vs_l, vs_r, src2d, dst2d)
    vs2_l, vs2_r = _mid(vs_l, vs_r, acc1_l, acc1_r, dis,
                        W1, b1.reshape(1, -1), W2)
    acc2_l, acc2_r = _edge_agg_kernel()(vs2_l, vs2_r, src2d, dst2d)
    return _post(vs2_l, vs2_r, acc2_l, acc2_r, dis, b2.reshape(1, -1))
